# baseline (speedup 1.0000x reference)
import jax
import jax.numpy as jnp
from jax.experimental import pallas as pl
from jax.experimental.pallas import tpu as pltpu


_N_CHUNKS = 64


def _copy_body(x_ref, o_ref, sems):
    rows = x_ref.shape[0]
    chunk = rows // _N_CHUNKS
    copies = [
        pltpu.make_async_copy(
            x_ref.at[pl.ds(i * chunk, chunk)],
            o_ref.at[pl.ds(i * chunk, chunk)],
            sems.at[i],
        )
        for i in range(_N_CHUNKS)
    ]
    for c in copies:
        c.start()
    for c in copies:
        c.wait()


def kernel(inputs_embeds):
    shape = inputs_embeds.shape
    x2d = inputs_embeds.reshape(-1, shape[-1])
    out = pl.pallas_call(
        _copy_body,
        out_shape=jax.ShapeDtypeStruct(x2d.shape, x2d.dtype),
        in_specs=[pl.BlockSpec(memory_space=pl.ANY)],
        out_specs=pl.BlockSpec(memory_space=pl.ANY),
        scratch_shapes=[pltpu.SemaphoreType.DMA((_N_CHUNKS,))],
    )(x2d)
    return (out.reshape(shape), jnp.float32(0.0))


# speedup vs baseline: 46.8437x; 46.8437x over previous
import jax
import jax.numpy as jnp
from jax.experimental import pallas as pl
from jax.experimental.pallas import tpu as pltpu


_CHUNK_ROWS = 2048
_SLOTS = 8
_LOOKAHEAD = 4


def _copy_body(x_ref, o_ref, buf, in_sems, out_sems):
    n = x_ref.shape[0] // _CHUNK_ROWS

    def in_copy(i):
        s = i % _SLOTS
        return pltpu.make_async_copy(
            x_ref.at[pl.ds(i * _CHUNK_ROWS, _CHUNK_ROWS)], buf.at[s], in_sems.at[s]
        )

    def out_copy(i):
        s = i % _SLOTS
        return pltpu.make_async_copy(
            buf.at[s], o_ref.at[pl.ds(i * _CHUNK_ROWS, _CHUNK_ROWS)], out_sems.at[s]
        )

    for j in range(min(_LOOKAHEAD, n)):
        in_copy(j).start()
    for i in range(n):
        p = i + _LOOKAHEAD
        if p < n:
            if p - _SLOTS >= 0:
                out_copy(p - _SLOTS).wait()
            in_copy(p).start()
        in_copy(i).wait()
        out_copy(i).start()
    for i in range(max(0, n - _SLOTS), n):
        out_copy(i).wait()


def kernel(inputs_embeds):
    shape = inputs_embeds.shape
    x2d = inputs_embeds.reshape(-1, shape[-1])
    cols = x2d.shape[1]
    out = pl.pallas_call(
        _copy_body,
        out_shape=jax.ShapeDtypeStruct(x2d.shape, x2d.dtype),
        in_specs=[pl.BlockSpec(memory_space=pl.ANY)],
        out_specs=pl.BlockSpec(memory_space=pl.ANY),
        scratch_shapes=[
            pltpu.VMEM((_SLOTS, _CHUNK_ROWS, cols), x2d.dtype),
            pltpu.SemaphoreType.DMA((_SLOTS,)),
            pltpu.SemaphoreType.DMA((_SLOTS,)),
        ],
    )(x2d)
    return (out.reshape(shape), jnp.float32(0.0))
